# baseline (speedup 1.0000x reference)
import functools

import jax
import jax.numpy as jnp
from jax import lax
from jax.experimental import pallas as pl
from jax.experimental.pallas import tpu as pltpu
from jax.experimental.pallas import tpu_sc as plsc

LANES = 16
NC, NS = 2, 16
NW = NC * NS

B, V, L, D = 8, 8, 4096, 128
SEGS = B * V
SEGS_PER_W = SEGS // NW

L_SC = 1024
L_TC = L - L_SC

CHUNK = 256
NCHUNK = L_SC // CHUNK
ROW_UNROLL = 4
DV = D // LANES

TC_BLK = 1024
TC_NBLK = L_TC // TC_BLK


def _sc_body(x_hbm, out_hbm, buf0, buf1, outv, sem0, sem1):
    wid = lax.axis_index("s") * NC + lax.axis_index("c")
    base_seg = wid * SEGS_PER_W
    bufs = (buf0, buf1)
    sems = (sem0, sem1)

    def start(g):
        seg = base_seg + (g // NCHUNK)
        row0 = (g % NCHUNK) * CHUNK
        return pltpu.async_copy(
            x_hbm.at[seg, pl.ds(row0, CHUNK)], bufs[g % 2], sems[g % 2]
        )

    total = SEGS_PER_W * NCHUNK
    handle = start(0)
    acc = tuple(jnp.zeros((LANES,), jnp.float32) for _ in range(DV))

    for g in range(total):
        next_handle = start(g + 1) if g + 1 < total else None
        handle.wait()
        buf = bufs[g % 2]

        def body(i, a, buf=buf):
            r = i * ROW_UNROLL
            out = list(a)
            for k in range(ROW_UNROLL):
                for j in range(DV):
                    out[j] = out[j] + buf[r + k, pl.ds(j * LANES, LANES)]
            return tuple(out)

        acc = lax.fori_loop(0, CHUNK // ROW_UNROLL, body, acc)

        if (g + 1) % NCHUNK == 0:
            s = g // NCHUNK
            for j in range(DV):
                outv[s, pl.ds(j * LANES, LANES)] = acc[j] * (1.0 / float(D))
            acc = tuple(jnp.zeros((LANES,), jnp.float32) for _ in range(DV))
        handle = next_handle

    pltpu.sync_copy(outv, out_hbm.at[pl.ds(base_seg, SEGS_PER_W)])


TC_BLK0 = L_SC // TC_BLK


def _tc_body(x_ref, o_ref):
    j = pl.program_id(1)

    @pl.when(j == 0)
    def _():
        o_ref[...] = jnp.zeros_like(o_ref)

    o_ref[...] += jnp.sum(x_ref[...], axis=1, keepdims=True) * (1.0 / float(D))


@jax.jit
def kernel(X):
    xf = X.reshape(SEGS, L, D)

    sc_part = pl.kernel(
        _sc_body,
        out_type=jax.ShapeDtypeStruct((SEGS, D), jnp.float32),
        mesh=plsc.VectorSubcoreMesh(core_axis_name="c", subcore_axis_name="s"),
        scratch_types=[
            pltpu.VMEM((CHUNK, D), jnp.float32),
            pltpu.VMEM((CHUNK, D), jnp.float32),
            pltpu.VMEM((SEGS_PER_W, D), jnp.float32),
            pltpu.SemaphoreType.DMA,
            pltpu.SemaphoreType.DMA,
        ],
    )(xf)

    tc_part = pl.pallas_call(
        _tc_body,
        grid=(SEGS, TC_NBLK),
        in_specs=[
            pl.BlockSpec((1, TC_BLK, D), lambda i, j: (i, j + TC_BLK0, 0))
        ],
        out_specs=pl.BlockSpec((1, 1, D), lambda i, j: (i, 0, 0)),
        out_shape=jax.ShapeDtypeStruct((SEGS, 1, D), jnp.float32),
    )(xf)

    tc_part = tc_part.reshape(SEGS, D)

    ret = sc_part + tc_part
    ret = jnp.where(jnp.isnan(ret), jnp.zeros_like(ret), ret)
    return ret.reshape(B, V, D)


# speedup vs baseline: 1.9424x; 1.9424x over previous
import functools

import jax
import jax.numpy as jnp
from jax import lax
from jax.experimental import pallas as pl
from jax.experimental.pallas import tpu as pltpu
from jax.experimental.pallas import tpu_sc as plsc

LANES = 16
NC, NS = 2, 16
NW = NC * NS

B, V, L, D = 8, 8, 4096, 128
SEGS = B * V
SEGS_PER_W = SEGS // NW

L_SC = 1024
L_TC = L - L_SC

CHUNK = 256
NCHUNK = L_SC // CHUNK
ROW_UNROLL = 4
DV = D // LANES

TC_BLK = 1024
TC_NBLK = L_TC // TC_BLK
TC_NBUF = 2 * TC_NBLK


def _sc_body(x_hbm, out_hbm, buf0, buf1, outv, sem0, sem1):
    wid = lax.axis_index("s") * NC + lax.axis_index("c")
    base_seg = wid * SEGS_PER_W
    bufs = (buf0, buf1)
    sems = (sem0, sem1)

    def start(g):
        seg = base_seg + (g // NCHUNK)
        row0 = (g % NCHUNK) * CHUNK
        return pltpu.async_copy(
            x_hbm.at[seg, pl.ds(row0, CHUNK)], bufs[g % 2], sems[g % 2]
        )

    total = SEGS_PER_W * NCHUNK
    handle = start(0)
    acc = tuple(jnp.zeros((LANES,), jnp.float32) for _ in range(DV))

    for g in range(total):
        next_handle = start(g + 1) if g + 1 < total else None
        handle.wait()
        buf = bufs[g % 2]

        def body(i, a, buf=buf):
            r = i * ROW_UNROLL
            out = list(a)
            for k in range(ROW_UNROLL):
                for j in range(DV):
                    out[j] = out[j] + buf[r + k, pl.ds(j * LANES, LANES)]
            return tuple(out)

        acc = lax.fori_loop(0, CHUNK // ROW_UNROLL, body, acc)

        if (g + 1) % NCHUNK == 0:
            s = g // NCHUNK
            for j in range(DV):
                outv[s, pl.ds(j * LANES, LANES)] = acc[j] * (1.0 / float(D))
            acc = tuple(jnp.zeros((LANES,), jnp.float32) for _ in range(DV))
        handle = next_handle

    pltpu.sync_copy(outv, out_hbm.at[pl.ds(base_seg, SEGS_PER_W)])


def _tc_body(x_hbm, o_ref, bufs, sems):

    def copy(seg, c, phase):
        b = phase * TC_NBLK + c
        return pltpu.make_async_copy(
            x_hbm.at[seg, pl.ds(L_SC + c * TC_BLK, TC_BLK)],
            bufs.at[b],
            sems.at[b],
        )

    for c in range(TC_NBLK):
        copy(0, c, 0).start()
        copy(1, c, 1).start()

    def body(p, _):
        for k in range(2):
            seg = 2 * p + k
            acc = jnp.zeros((1, D), jnp.float32)
            for c in range(TC_NBLK):
                copy(seg, c, k).wait()
                acc = acc + jnp.sum(bufs[k * TC_NBLK + c], axis=0,
                                    keepdims=True)

                @pl.when(seg + 2 < SEGS)
                def _():
                    copy(seg + 2, c, k).start()

            o_ref[pl.ds(seg, 1), :] = acc * (1.0 / float(D))
        return 0

    lax.fori_loop(0, SEGS // 2, body, 0)


@jax.jit
def kernel(X):
    xf = X.reshape(SEGS, L, D)

    sc_part = pl.kernel(
        _sc_body,
        out_type=jax.ShapeDtypeStruct((SEGS, D), jnp.float32),
        mesh=plsc.VectorSubcoreMesh(core_axis_name="c", subcore_axis_name="s"),
        scratch_types=[
            pltpu.VMEM((CHUNK, D), jnp.float32),
            pltpu.VMEM((CHUNK, D), jnp.float32),
            pltpu.VMEM((SEGS_PER_W, D), jnp.float32),
            pltpu.SemaphoreType.DMA,
            pltpu.SemaphoreType.DMA,
        ],
    )(xf)

    tc_part = pl.pallas_call(
        _tc_body,
        in_specs=[pl.BlockSpec(memory_space=pl.ANY)],
        out_specs=pl.BlockSpec(memory_space=pltpu.VMEM),
        out_shape=jax.ShapeDtypeStruct((SEGS, D), jnp.float32),
        scratch_shapes=[
            pltpu.VMEM((TC_NBUF, TC_BLK, D), jnp.float32),
            pltpu.SemaphoreType.DMA((TC_NBUF,)),
        ],
    )(xf)

    ret = sc_part + tc_part
    ret = jnp.where(jnp.isnan(ret), jnp.zeros_like(ret), ret)
    return ret.reshape(B, V, D)


# speedup vs baseline: 2.1237x; 1.0933x over previous
import jax
import jax.numpy as jnp
from jax import lax
from jax.experimental import pallas as pl
from jax.experimental.pallas import tpu as pltpu
from jax.experimental.pallas import tpu_sc as plsc

LANES = 16
NC, NS = 2, 16
NW = NC * NS

B, V, L, D = 8, 8, 4096, 128
SEGS = B * V
SEGS_PER_W = SEGS // NW

L_SC = 1280
L_TC = L - L_SC

CHUNK = 256
NCHUNK = L_SC // CHUNK
ROW_UNROLL = 4
DV = D // LANES

TC_RING = 4


def _sc_body(x_hbm, out_hbm, buf, outv, sem):
    wid = lax.axis_index("s") * NC + lax.axis_index("c")
    base_seg = wid * SEGS_PER_W

    for s in range(SEGS_PER_W):
        seg = base_seg + s

        def start(g):
            return pltpu.async_copy(
                x_hbm.at[seg, pl.ds(g * CHUNK, CHUNK)],
                buf.at[pl.ds(lax.rem(g, 2) * CHUNK, CHUNK)],
                sem,
            )

        start(0)
        start(1)

        def chunk_body(g, acc):
            pltpu.make_async_copy(
                x_hbm.at[seg, pl.ds(0, CHUNK)],
                buf.at[pl.ds(0, CHUNK)],
                sem,
            ).wait()
            base = lax.rem(g, 2) * CHUNK

            def row_body(i, a):
                r = base + i * ROW_UNROLL
                out = list(a)
                for k in range(ROW_UNROLL):
                    for j in range(DV):
                        out[j] = out[j] + buf[r + k, pl.ds(j * LANES, LANES)]
                return tuple(out)

            acc = lax.fori_loop(0, CHUNK // ROW_UNROLL, row_body, acc)

            @pl.when(g + 2 < NCHUNK)
            def _():
                start(g + 2)

            return acc

        acc = tuple(jnp.zeros((LANES,), jnp.float32) for _ in range(DV))
        acc = lax.fori_loop(0, NCHUNK, chunk_body, acc)
        for j in range(DV):
            outv[s, pl.ds(j * LANES, LANES)] = acc[j] * (1.0 / float(D))

    pltpu.sync_copy(outv, out_hbm.at[pl.ds(base_seg, SEGS_PER_W)])


def _tc_body(x_hbm, o_ref, bufs, sems):
    def copy(seg, slot):
        return pltpu.make_async_copy(
            x_hbm.at[seg, pl.ds(L_SC, L_TC)], bufs.at[slot], sems.at[slot]
        )

    for k in range(TC_RING):
        copy(k, k).start()

    def body(p, _):
        for k in range(TC_RING):
            seg = TC_RING * p + k
            copy(seg, k).wait()
            acc = jnp.sum(bufs[k], axis=0, keepdims=True)
            o_ref[pl.ds(seg, 1), :] = acc * (1.0 / float(D))

            @pl.when(seg + TC_RING < SEGS)
            def _():
                copy(seg + TC_RING, k).start()

        return 0

    lax.fori_loop(0, SEGS // TC_RING, body, 0)


@jax.jit
def kernel(X):
    xf = X.reshape(SEGS, L, D)

    sc_part = pl.kernel(
        _sc_body,
        out_type=jax.ShapeDtypeStruct((SEGS, D), jnp.float32),
        mesh=plsc.VectorSubcoreMesh(core_axis_name="c", subcore_axis_name="s"),
        scratch_types=[
            pltpu.VMEM((2 * CHUNK, D), jnp.float32),
            pltpu.VMEM((SEGS_PER_W, D), jnp.float32),
            pltpu.SemaphoreType.DMA,
        ],
    )(xf)

    tc_part = pl.pallas_call(
        _tc_body,
        in_specs=[pl.BlockSpec(memory_space=pl.ANY)],
        out_specs=pl.BlockSpec(memory_space=pltpu.VMEM),
        out_shape=jax.ShapeDtypeStruct((SEGS, D), jnp.float32),
        scratch_shapes=[
            pltpu.VMEM((TC_RING, L_TC, D), jnp.float32),
            pltpu.SemaphoreType.DMA((TC_RING,)),
        ],
    )(xf)

    ret = sc_part + tc_part
    ret = jnp.where(jnp.isnan(ret), jnp.zeros_like(ret), ret)
    return ret.reshape(B, V, D)
